# baseline (speedup 1.0000x reference)
import functools

import jax
import jax.numpy as jnp
from jax import lax
from jax.experimental import pallas as pl
from jax.experimental.pallas import tpu as pltpu
from jax.experimental.pallas import tpu_sc as plsc

N = 10000
D = 128
E = 320000
G = 64
HALF = 64

NC = 2
NS = 16
K = 128
NBUF = 4
NCHUNK = 160
NGRP = NCHUNK // NBUF
EPT = NCHUNK * K
RPT = N // NS
ACC_ROWS = N + 16
PQ = 5
PK = RPT // PQ

_mesh = plsc.VectorSubcoreMesh(core_axis_name="c", subcore_axis_name="s")
_sc_params = pltpu.CompilerParams(use_tc_tiling_on_sc=False)


def _agg_impl(do_pool, h_hbm, src_hbm, dst_hbm, *rest):
    if do_pool:
        (batch_hbm, zero_hbm, pool_out,
         acc, idx_s, idx_d, rows, stage, pooled, bidx, sem_g, sem_s) = rest
    else:
        (out_hbm, acc, idx_s, idx_d, rows, stage, sem_g, sem_s) = rest
    c = lax.axis_index("c")
    s = lax.axis_index("s")
    base = s * RPT

    pltpu.sync_copy(src_hbm.at[c, s], idx_s)
    pltpu.sync_copy(dst_hbm.at[s], idx_d)

    for q in range(PQ):
        pltpu.sync_copy(h_hbm.at[pl.ds(c * N + base + q * PK, PK)], stage)
        pltpu.sync_copy(stage, acc.at[pl.ds(base + q * PK, PK)])
    plsc.subcore_barrier()

    for b in range(NBUF):
        pltpu.async_copy(h_hbm.at[idx_s.at[b]], rows.at[b], sem_g)

    def body(i, carry):
        for b in range(NBUF):
            j = i * NBUF + b
            pltpu.make_async_copy(h_hbm.at[idx_s.at[j]], rows.at[b],
                                  sem_g).wait()
            pltpu.async_copy(rows.at[b], acc.at[idx_d.at[j]], sem_s, add=True)

        @pl.when(i < NGRP - 1)
        def _():
            for b in range(NBUF):
                j = i * NBUF + b
                pltpu.make_async_copy(rows.at[b], acc.at[idx_d.at[j]],
                                      sem_s).wait()
                pltpu.async_copy(h_hbm.at[idx_s.at[j + NBUF]], rows.at[b],
                                 sem_g)
        return carry

    lax.fori_loop(0, NGRP, body, 0)
    for b in range(NBUF):
        pltpu.make_async_copy(rows.at[b], acc.at[idx_d.at[0]], sem_s).wait()
    plsc.subcore_barrier()

    if not do_pool:
        for q in range(PQ):
            pltpu.sync_copy(acc.at[pl.ds(base + q * PK, PK)], stage)
            pltpu.sync_copy(stage, out_hbm.at[pl.ds(c * N + base + q * PK, PK)])
        return

    @pl.when(s == 0)
    def _():
        pltpu.sync_copy(zero_hbm, pooled)
    plsc.subcore_barrier()
    pltpu.sync_copy(batch_hbm.at[s], bidx)
    for q in range(PQ):
        pltpu.sync_copy(acc.at[pl.ds(base + q * PK, PK)], stage)
        pltpu.sync_copy(stage, pooled.at[bidx.at[q]], add=True)
    plsc.subcore_barrier()

    @pl.when(s == 0)
    def _():
        pltpu.sync_copy(pooled, stage.at[pl.ds(0, G)])
        pltpu.sync_copy(stage.at[pl.ds(0, G)], pool_out.at[c])


_agg = functools.partial(
    pl.kernel,
    functools.partial(_agg_impl, False),
    out_type=jax.ShapeDtypeStruct((2 * N, HALF), jnp.float32),
    mesh=_mesh,
    scratch_types=[
        pltpu.VMEM_SHARED((ACC_ROWS, HALF), jnp.float32),
        pltpu.VMEM((NCHUNK, K), jnp.int32),
        pltpu.VMEM((NCHUNK, K), jnp.int32),
        pltpu.VMEM((NBUF, K, HALF), jnp.float32),
        pltpu.VMEM((PK, HALF), jnp.float32),
        pltpu.SemaphoreType.DMA,
        pltpu.SemaphoreType.DMA,
    ],
    compiler_params=_sc_params,
)()

_agg_pool = functools.partial(
    pl.kernel,
    functools.partial(_agg_impl, True),
    out_type=jax.ShapeDtypeStruct((NC, G, HALF), jnp.float32),
    mesh=_mesh,
    scratch_types=[
        pltpu.VMEM_SHARED((ACC_ROWS, HALF), jnp.float32),
        pltpu.VMEM((NCHUNK, K), jnp.int32),
        pltpu.VMEM((NCHUNK, K), jnp.int32),
        pltpu.VMEM((NBUF, K, HALF), jnp.float32),
        pltpu.VMEM((PK, HALF), jnp.float32),
        pltpu.VMEM_SHARED((G, HALF), jnp.float32),
        pltpu.VMEM((PQ, PK), jnp.int32),
        pltpu.SemaphoreType.DMA,
        pltpu.SemaphoreType.DMA,
    ],
    compiler_params=_sc_params,
)()


BLK = 2000


def _mm_body(h0_ref, h1_ref, wt_ref, b_ref, o_ref):
    h0 = h0_ref[...]
    h1 = h1_ref[...]
    wt = wt_ref[0]
    acc = lax.dot_general(h0, wt[:HALF], (((1,), (0,)), ((), ())),
                          preferred_element_type=jnp.float32)
    acc += lax.dot_general(h1, wt[HALF:], (((1,), (0,)), ((), ())),
                           preferred_element_type=jnp.float32)
    o_ref[...] = jnp.maximum(acc + b_ref[0], 0.0)


_mm = pl.pallas_call(
    _mm_body,
    grid=(2, N // BLK),
    in_specs=[
        pl.BlockSpec((BLK, HALF), lambda half, i: (i, 0)),
        pl.BlockSpec((BLK, HALF), lambda half, i: (N // BLK + i, 0)),
        pl.BlockSpec((1, D, HALF), lambda half, i: (half, 0, 0)),
        pl.BlockSpec((1, 1, HALF), lambda half, i: (half, 0, 0)),
    ],
    out_specs=pl.BlockSpec((BLK, HALF), lambda half, i: (half * (N // BLK) + i, 0)),
    out_shape=jax.ShapeDtypeStruct((2 * N, HALF), jnp.float32),
)


def _prep_edges(ei):
    src, dst = ei[0], ei[1]
    pad = NS * NCHUNK * K - E
    src = jnp.concatenate([src, jnp.zeros((pad,), jnp.int32)])
    dst = jnp.concatenate([dst, jnp.full((pad,), N, jnp.int32)])
    src = src.reshape(NS, NCHUNK, K)
    src2 = src[None] + (jnp.arange(NC, dtype=jnp.int32) * N)[:, None, None, None]
    return src2, dst.reshape(NS, NCHUNK, K)


def kernel(x, edge_index, expander_edge_index, batch, W1, b1, W2, b2, W3, b3):
    h = jnp.concatenate([x[:, :HALF], x[:, HALF:]], axis=0)
    src_e, dst_e = _prep_edges(edge_index)
    src_x, dst_x = _prep_edges(expander_edge_index)
    batch_i = batch.reshape(NS, PQ, PK)
    zero = jnp.zeros((G, HALF), jnp.float32)
    for li, (W, b) in enumerate(((W1, b1), (W2, b2), (W3, b3))):
        h = _agg(h, src_e, dst_e)
        wt = W.T.reshape(1, D, D)
        wt = jnp.concatenate([wt[:, :, :HALF], wt[:, :, HALF:]], axis=0)
        h = _mm(h, h, wt, b.reshape(2, 1, HALF))
        if li < 2:
            h = _agg(h, src_x, dst_x)
        else:
            pooled = _agg_pool(h, src_x, dst_x, batch_i, zero)
    return jnp.concatenate([pooled[0], pooled[1]], axis=1).reshape(-1)


# speedup vs baseline: 2.0078x; 2.0078x over previous
import functools

import jax
import jax.numpy as jnp
from jax import lax
from jax.experimental import pallas as pl
from jax.experimental.pallas import tpu as pltpu
from jax.experimental.pallas import tpu_sc as plsc

N = 10000
D = 128
E = 320000
G = 64
HALF = 64

NC = 2
NS = 16
K = 128
NIDX = 8
NBLK = 20
NCHUNK = NBLK * NIDX
NGRP = NCHUNK // 2
EPT = NCHUNK * K
RPT = N // NS
ACC_ROWS = N + 16
PQ = 5
PK = RPT // PQ

_mesh = plsc.VectorSubcoreMesh(core_axis_name="c", subcore_axis_name="s")
_sc_params = pltpu.CompilerParams(use_tc_tiling_on_sc=False)


def _agg_impl(do_pool, h_hbm, src_hbm, dst_hbm, *rest):
    if do_pool:
        (batch_hbm, zero_hbm, pool_out, acc, h_spm,
         idx_s, idx_d, rows, stage, pooled, bidx, sem_g, sem_s, sem_i) = rest
    else:
        (out_hbm, acc, h_spm,
         idx_s, idx_d, rows, stage, sem_g, sem_s, sem_i) = rest
    c = lax.axis_index("c")
    s = lax.axis_index("s")
    base = s * RPT

    for q in range(PQ):
        pltpu.sync_copy(h_hbm.at[pl.ds(c * N + base + q * PK, PK)], stage)
        pltpu.sync_copy(stage, acc.at[pl.ds(base + q * PK, PK)])
        pltpu.sync_copy(stage, h_spm.at[pl.ds(base + q * PK, PK)])
    for blk in range(2):
        pltpu.async_copy(src_hbm.at[s, blk], idx_s.at[blk], sem_i)
        pltpu.async_copy(dst_hbm.at[s, blk], idx_d.at[blk], sem_i)
    plsc.subcore_barrier()

    def body(i, carry):
        @pl.when(i % 4 == 0)
        def _():
            pltpu.make_async_copy(src_hbm.at[s, 0], idx_s.at[0],
                                  sem_i).wait()
            pltpu.make_async_copy(dst_hbm.at[s, 0], idx_d.at[0],
                                  sem_i).wait()

        @pl.when(jnp.logical_and(i % 4 == 1, i // 4 + 2 < NBLK))
        def _():
            b2 = i // 4 + 2
            slot2 = b2 % 3
            pltpu.async_copy(src_hbm.at[s, b2], idx_s.at[slot2], sem_i)
            pltpu.async_copy(dst_hbm.at[s, b2], idx_d.at[slot2], sem_i)

        for u in range(2):
            j = 2 * i + u
            blk = j // NIDX
            slot = blk % 3
            cb = j % NIDX

            @pl.when(j >= 2)
            def _():
                pltpu.make_async_copy(rows.at[u], acc.at[idx_d.at[0, 0]],
                                      sem_s).wait()

            pltpu.async_copy(h_spm.at[idx_s.at[slot, cb]], rows.at[u],
                             sem_g).wait()
            pltpu.async_copy(rows.at[u], acc.at[idx_d.at[slot, cb]],
                             sem_s, add=True)
        return carry

    lax.fori_loop(0, NGRP, body, 0)
    for u in range(2):
        pltpu.make_async_copy(rows.at[u], acc.at[idx_d.at[0, 0]],
                              sem_s).wait()
    plsc.subcore_barrier()

    if not do_pool:
        for q in range(PQ):
            pltpu.sync_copy(acc.at[pl.ds(base + q * PK, PK)], stage)
            pltpu.sync_copy(stage, out_hbm.at[pl.ds(c * N + base + q * PK, PK)])
        return

    @pl.when(s == 0)
    def _():
        pltpu.sync_copy(zero_hbm, pooled)
    plsc.subcore_barrier()
    pltpu.sync_copy(batch_hbm.at[s], bidx)
    for q in range(PQ):
        pltpu.sync_copy(acc.at[pl.ds(base + q * PK, PK)], stage)
        pltpu.sync_copy(stage, pooled.at[bidx.at[q]], add=True)
    plsc.subcore_barrier()

    @pl.when(s == 0)
    def _():
        pltpu.sync_copy(pooled, stage.at[pl.ds(0, G)])
        pltpu.sync_copy(stage.at[pl.ds(0, G)], pool_out.at[c])


_agg = functools.partial(
    pl.kernel,
    functools.partial(_agg_impl, False),
    out_type=jax.ShapeDtypeStruct((2 * N, HALF), jnp.float32),
    mesh=_mesh,
    scratch_types=[
        pltpu.VMEM_SHARED((ACC_ROWS, HALF), jnp.float32),
        pltpu.VMEM_SHARED((N, HALF), jnp.float32),
        pltpu.VMEM((3, NIDX, K), jnp.int32),
        pltpu.VMEM((3, NIDX, K), jnp.int32),
        pltpu.VMEM((2, K, HALF), jnp.float32),
        pltpu.VMEM((PK, HALF), jnp.float32),
        pltpu.SemaphoreType.DMA,
        pltpu.SemaphoreType.DMA,
        pltpu.SemaphoreType.DMA,
    ],
    compiler_params=_sc_params,
)()

_agg_pool = functools.partial(
    pl.kernel,
    functools.partial(_agg_impl, True),
    out_type=jax.ShapeDtypeStruct((NC, G, HALF), jnp.float32),
    mesh=_mesh,
    scratch_types=[
        pltpu.VMEM_SHARED((ACC_ROWS, HALF), jnp.float32),
        pltpu.VMEM_SHARED((N, HALF), jnp.float32),
        pltpu.VMEM((3, NIDX, K), jnp.int32),
        pltpu.VMEM((3, NIDX, K), jnp.int32),
        pltpu.VMEM((2, K, HALF), jnp.float32),
        pltpu.VMEM((PK, HALF), jnp.float32),
        pltpu.VMEM_SHARED((G, HALF), jnp.float32),
        pltpu.VMEM((PQ, PK), jnp.int32),
        pltpu.SemaphoreType.DMA,
        pltpu.SemaphoreType.DMA,
        pltpu.SemaphoreType.DMA,
    ],
    compiler_params=_sc_params,
)()


BLK = 2000


def _mm_body(h0_ref, h1_ref, wt_ref, b_ref, o_ref):
    h0 = h0_ref[...]
    h1 = h1_ref[...]
    wt = wt_ref[0]
    acc = lax.dot_general(h0, wt[:HALF], (((1,), (0,)), ((), ())),
                          preferred_element_type=jnp.float32)
    acc += lax.dot_general(h1, wt[HALF:], (((1,), (0,)), ((), ())),
                           preferred_element_type=jnp.float32)
    o_ref[...] = jnp.maximum(acc + b_ref[0], 0.0)


_mm = pl.pallas_call(
    _mm_body,
    grid=(2, N // BLK),
    in_specs=[
        pl.BlockSpec((BLK, HALF), lambda half, i: (i, 0)),
        pl.BlockSpec((BLK, HALF), lambda half, i: (N // BLK + i, 0)),
        pl.BlockSpec((1, D, HALF), lambda half, i: (half, 0, 0)),
        pl.BlockSpec((1, 1, HALF), lambda half, i: (half, 0, 0)),
    ],
    out_specs=pl.BlockSpec((BLK, HALF), lambda half, i: (half * (N // BLK) + i, 0)),
    out_shape=jax.ShapeDtypeStruct((2 * N, HALF), jnp.float32),
)


def _prep_edges(ei):
    src, dst = ei[0], ei[1]
    pad = NS * NCHUNK * K - E
    src = jnp.concatenate([src, jnp.zeros((pad,), jnp.int32)])
    dst = jnp.concatenate([dst, jnp.full((pad,), N, jnp.int32)])
    return (src.reshape(NS, NBLK, NIDX, K), dst.reshape(NS, NBLK, NIDX, K))


def kernel(x, edge_index, expander_edge_index, batch, W1, b1, W2, b2, W3, b3):
    h = jnp.concatenate([x[:, :HALF], x[:, HALF:]], axis=0)
    src_e, dst_e = _prep_edges(edge_index)
    src_x, dst_x = _prep_edges(expander_edge_index)
    batch_i = batch.reshape(NS, PQ, PK)
    zero = jnp.zeros((G, HALF), jnp.float32)
    for li, (W, b) in enumerate(((W1, b1), (W2, b2), (W3, b3))):
        h = _agg(h, src_e, dst_e)
        wt = W.T.reshape(1, D, D)
        wt = jnp.concatenate([wt[:, :, :HALF], wt[:, :, HALF:]], axis=0)
        h = _mm(h, h, wt, b.reshape(2, 1, HALF))
        if li < 2:
            h = _agg(h, src_x, dst_x)
        else:
            pooled = _agg_pool(h, src_x, dst_x, batch_i, zero)
    return jnp.concatenate([pooled[0], pooled[1]], axis=1).reshape(-1)


# speedup vs baseline: 2.2308x; 1.1111x over previous
import functools

import jax
import jax.numpy as jnp
from jax import lax
from jax.experimental import pallas as pl
from jax.experimental.pallas import tpu as pltpu
from jax.experimental.pallas import tpu_sc as plsc

N = 10000
D = 128
E = 320000
G = 64
HALF = 64

NC = 2
NS = 16
K = 128
NIDX = 8
NBLK = 20
NCHUNK = NBLK * NIDX
NGRP = NCHUNK // 2
EPT = NCHUNK * K
RPT = N // NS
ACC_ROWS = N + 16
PQ = 5
PK = RPT // PQ

_mesh = plsc.VectorSubcoreMesh(core_axis_name="c", subcore_axis_name="s")
_sc_params = pltpu.CompilerParams(use_tc_tiling_on_sc=False)


def _agg_impl(do_pool, h_hbm, src_hbm, dst_hbm, *rest):
    if do_pool:
        (batch_hbm, zero_hbm, pool_out, acc, h_spm,
         idx_s, idx_d, rows, stage, pooled, bidx, sem_g, sem_s, sem_i) = rest
    else:
        (out_hbm, acc, h_spm,
         idx_s, idx_d, rows, stage, sem_g, sem_s, sem_i) = rest
    c = lax.axis_index("c")
    s = lax.axis_index("s")
    base = s * RPT

    d1 = pltpu.async_copy(h_hbm.at[pl.ds(c * N + base, RPT)],
                          acc.at[pl.ds(base, RPT)], sem_i)
    d2 = pltpu.async_copy(h_hbm.at[pl.ds(c * N + base, RPT)],
                          h_spm.at[pl.ds(base, RPT)], sem_i)
    for blk in range(2):
        pltpu.async_copy(src_hbm.at[s, blk], idx_s.at[blk], sem_i)
        pltpu.async_copy(dst_hbm.at[s, blk], idx_d.at[blk], sem_i)
    d1.wait()
    d2.wait()
    plsc.subcore_barrier()

    def body(i, carry):
        @pl.when(i % 2 == 0)
        def _():
            pltpu.make_async_copy(src_hbm.at[s, 0], idx_s.at[0],
                                  sem_i).wait()
            pltpu.make_async_copy(dst_hbm.at[s, 0], idx_d.at[0],
                                  sem_i).wait()

        @pl.when(jnp.logical_and(i % 2 == 1, i // 2 + 2 < NBLK))
        def _():
            b2 = i // 2 + 2
            slot2 = b2 % 3
            pltpu.async_copy(src_hbm.at[s, b2], idx_s.at[slot2], sem_i)
            pltpu.async_copy(dst_hbm.at[s, b2], idx_d.at[slot2], sem_i)

        descs = []
        for u in range(4):
            j = 4 * i + u
            slot = (j // NIDX) % 3
            cb = j % NIDX

            @pl.when(i > 0)
            def _():
                pltpu.make_async_copy(rows.at[u], acc.at[idx_d.at[0, 0]],
                                      sem_s).wait()

            descs.append(pltpu.async_copy(h_spm.at[idx_s.at[slot, cb]],
                                          rows.at[u], sem_g))
        for u in range(4):
            j = 4 * i + u
            slot = (j // NIDX) % 3
            cb = j % NIDX
            descs[u].wait()
            pltpu.async_copy(rows.at[u], acc.at[idx_d.at[slot, cb]],
                             sem_s, add=True)
        return carry

    lax.fori_loop(0, NCHUNK // 4, body, 0)
    for u in range(4):
        pltpu.make_async_copy(rows.at[u], acc.at[idx_d.at[0, 0]],
                              sem_s).wait()
    plsc.subcore_barrier()

    if not do_pool:
        pltpu.sync_copy(acc.at[pl.ds(base, RPT)],
                        out_hbm.at[pl.ds(c * N + base, RPT)])
        return

    @pl.when(s == 0)
    def _():
        pltpu.sync_copy(zero_hbm, pooled)
    plsc.subcore_barrier()
    pltpu.sync_copy(batch_hbm.at[s], bidx)
    for q in range(PQ):
        pltpu.sync_copy(acc.at[pl.ds(base + q * PK, PK)], stage)
        pltpu.sync_copy(stage, pooled.at[bidx.at[q]], add=True)
    plsc.subcore_barrier()

    @pl.when(s == 0)
    def _():
        pltpu.sync_copy(pooled, stage.at[pl.ds(0, G)])
        pltpu.sync_copy(stage.at[pl.ds(0, G)], pool_out.at[c])


_agg = functools.partial(
    pl.kernel,
    functools.partial(_agg_impl, False),
    out_type=jax.ShapeDtypeStruct((2 * N, HALF), jnp.float32),
    mesh=_mesh,
    scratch_types=[
        pltpu.VMEM_SHARED((ACC_ROWS, HALF), jnp.float32),
        pltpu.VMEM_SHARED((N, HALF), jnp.float32),
        pltpu.VMEM((3, NIDX, K), jnp.int32),
        pltpu.VMEM((3, NIDX, K), jnp.int32),
        pltpu.VMEM((4, K, HALF), jnp.float32),
        pltpu.VMEM((PK, HALF), jnp.float32),
        pltpu.SemaphoreType.DMA,
        pltpu.SemaphoreType.DMA,
        pltpu.SemaphoreType.DMA,
    ],
    compiler_params=_sc_params,
)()

_agg_pool = functools.partial(
    pl.kernel,
    functools.partial(_agg_impl, True),
    out_type=jax.ShapeDtypeStruct((NC, G, HALF), jnp.float32),
    mesh=_mesh,
    scratch_types=[
        pltpu.VMEM_SHARED((ACC_ROWS, HALF), jnp.float32),
        pltpu.VMEM_SHARED((N, HALF), jnp.float32),
        pltpu.VMEM((3, NIDX, K), jnp.int32),
        pltpu.VMEM((3, NIDX, K), jnp.int32),
        pltpu.VMEM((4, K, HALF), jnp.float32),
        pltpu.VMEM((PK, HALF), jnp.float32),
        pltpu.VMEM_SHARED((G, HALF), jnp.float32),
        pltpu.VMEM((PQ, PK), jnp.int32),
        pltpu.SemaphoreType.DMA,
        pltpu.SemaphoreType.DMA,
        pltpu.SemaphoreType.DMA,
    ],
    compiler_params=_sc_params,
)()


BLK = 2000


def _mm_body(h0_ref, h1_ref, wt_ref, b_ref, o_ref):
    h0 = h0_ref[...]
    h1 = h1_ref[...]
    wt = wt_ref[0]
    acc = lax.dot_general(h0, wt[:HALF], (((1,), (0,)), ((), ())),
                          preferred_element_type=jnp.float32)
    acc += lax.dot_general(h1, wt[HALF:], (((1,), (0,)), ((), ())),
                           preferred_element_type=jnp.float32)
    o_ref[...] = jnp.maximum(acc + b_ref[0], 0.0)


_mm = pl.pallas_call(
    _mm_body,
    grid=(2, N // BLK),
    in_specs=[
        pl.BlockSpec((BLK, HALF), lambda half, i: (i, 0)),
        pl.BlockSpec((BLK, HALF), lambda half, i: (N // BLK + i, 0)),
        pl.BlockSpec((1, D, HALF), lambda half, i: (half, 0, 0)),
        pl.BlockSpec((1, 1, HALF), lambda half, i: (half, 0, 0)),
    ],
    out_specs=pl.BlockSpec((BLK, HALF), lambda half, i: (half * (N // BLK) + i, 0)),
    out_shape=jax.ShapeDtypeStruct((2 * N, HALF), jnp.float32),
)


def _prep_edges(ei):
    src, dst = ei[0], ei[1]
    pad = NS * NCHUNK * K - E
    src = jnp.concatenate([src, jnp.zeros((pad,), jnp.int32)])
    dst = jnp.concatenate([dst, jnp.full((pad,), N, jnp.int32)])
    return (src.reshape(NS, NBLK, NIDX, K), dst.reshape(NS, NBLK, NIDX, K))


def kernel(x, edge_index, expander_edge_index, batch, W1, b1, W2, b2, W3, b3):
    h = jnp.concatenate([x[:, :HALF], x[:, HALF:]], axis=0)
    src_e, dst_e = _prep_edges(edge_index)
    src_x, dst_x = _prep_edges(expander_edge_index)
    batch_i = batch.reshape(NS, PQ, PK)
    zero = jnp.zeros((G, HALF), jnp.float32)
    for li, (W, b) in enumerate(((W1, b1), (W2, b2), (W3, b3))):
        h = _agg(h, src_e, dst_e)
        wt = W.T.reshape(1, D, D)
        wt = jnp.concatenate([wt[:, :, :HALF], wt[:, :, HALF:]], axis=0)
        h = _mm(h, h, wt, b.reshape(2, 1, HALF))
        if li < 2:
            h = _agg(h, src_x, dst_x)
        else:
            pooled = _agg_pool(h, src_x, dst_x, batch_i, zero)
    return jnp.concatenate([pooled[0], pooled[1]], axis=1).reshape(-1)


# speedup vs baseline: 2.2783x; 1.0213x over previous
import functools

import jax
import jax.numpy as jnp
from jax import lax
from jax.experimental import pallas as pl
from jax.experimental.pallas import tpu as pltpu
from jax.experimental.pallas import tpu_sc as plsc

N = 10000
D = 128
E = 320000
G = 64
HALF = 64

NC = 2
NS = 16
K = 128
NIDX = 8
NBLK = 20
NCHUNK = NBLK * NIDX
NGRP = NCHUNK // 2
EPT = NCHUNK * K
RPT = N // NS
ACC_ROWS = N + 16
PQ = 5
PK = RPT // PQ

_mesh = plsc.VectorSubcoreMesh(core_axis_name="c", subcore_axis_name="s")
_sc_params = pltpu.CompilerParams(use_tc_tiling_on_sc=False)


def _edge_phase(src_hbm, dst_hbm, s, gbuf, acc,
                idx_s, idx_d, rows, sem_g, sem_s, sem_i):
    for blk in range(2):
        pltpu.async_copy(src_hbm.at[s, blk], idx_s.at[blk], sem_i)
        pltpu.async_copy(dst_hbm.at[s, blk], idx_d.at[blk], sem_i)
    plsc.subcore_barrier()

    def body(i, carry):
        @pl.when(i % 2 == 0)
        def _():
            pltpu.make_async_copy(src_hbm.at[s, 0], idx_s.at[0],
                                  sem_i).wait()
            pltpu.make_async_copy(dst_hbm.at[s, 0], idx_d.at[0],
                                  sem_i).wait()

        @pl.when(jnp.logical_and(i % 2 == 1, i // 2 + 2 < NBLK))
        def _():
            b2 = i // 2 + 2
            slot2 = b2 % 3
            pltpu.async_copy(src_hbm.at[s, b2], idx_s.at[slot2], sem_i)
            pltpu.async_copy(dst_hbm.at[s, b2], idx_d.at[slot2], sem_i)

        descs = []
        for u in range(4):
            j = 4 * i + u
            slot = (j // NIDX) % 3
            cb = j % NIDX

            @pl.when(i > 0)
            def _():
                pltpu.make_async_copy(rows.at[u], acc.at[idx_d.at[0, 0]],
                                      sem_s).wait()

            descs.append(pltpu.async_copy(gbuf.at[idx_s.at[slot, cb]],
                                          rows.at[u], sem_g))
        for u in range(4):
            j = 4 * i + u
            slot = (j // NIDX) % 3
            cb = j % NIDX
            descs[u].wait()
            pltpu.async_copy(rows.at[u], acc.at[idx_d.at[slot, cb]],
                             sem_s, add=True)
        return carry

    lax.fori_loop(0, NCHUNK // 4, body, 0)
    for u in range(4):
        pltpu.make_async_copy(rows.at[u], acc.at[idx_d.at[0, 0]],
                              sem_s).wait()


def _agg_impl(do_pool, h_hbm, src_hbm, dst_hbm, *rest):
    if do_pool:
        (batch_hbm, zero_hbm, pool_out, acc, h_spm,
         idx_s, idx_d, rows, stage, pooled, bidx, sem_g, sem_s, sem_i) = rest
    else:
        (out_hbm, acc, h_spm,
         idx_s, idx_d, rows, stage, sem_g, sem_s, sem_i) = rest
    c = lax.axis_index("c")
    s = lax.axis_index("s")
    base = s * RPT

    d1 = pltpu.async_copy(h_hbm.at[pl.ds(c * N + base, RPT)],
                          acc.at[pl.ds(base, RPT)], sem_i)
    d2 = pltpu.async_copy(h_hbm.at[pl.ds(c * N + base, RPT)],
                          h_spm.at[pl.ds(base, RPT)], sem_i)
    d1.wait()
    d2.wait()
    _edge_phase(src_hbm, dst_hbm, s, h_spm, acc,
                idx_s, idx_d, rows, sem_g, sem_s, sem_i)
    plsc.subcore_barrier()

    if not do_pool:
        pltpu.sync_copy(acc.at[pl.ds(base, RPT)],
                        out_hbm.at[pl.ds(c * N + base, RPT)])
        return

    @pl.when(s == 0)
    def _():
        pltpu.sync_copy(zero_hbm, pooled)
    plsc.subcore_barrier()
    pltpu.sync_copy(batch_hbm.at[s], bidx)
    for q in range(PQ):
        pltpu.sync_copy(acc.at[pl.ds(base + q * PK, PK)], stage)
        pltpu.sync_copy(stage, pooled.at[bidx.at[q]], add=True)
    plsc.subcore_barrier()

    @pl.when(s == 0)
    def _():
        pltpu.sync_copy(pooled, stage.at[pl.ds(0, G)])
        pltpu.sync_copy(stage.at[pl.ds(0, G)], pool_out.at[c])


def _agg2_impl(h_hbm, srcA_hbm, dstA_hbm, srcB_hbm, dstB_hbm, out_hbm,
               bufA, bufB, idx_s, idx_d, rows, stage, sem_g, sem_s, sem_i):
    c = lax.axis_index("c")
    s = lax.axis_index("s")
    base = s * RPT

    d1 = pltpu.async_copy(h_hbm.at[pl.ds(c * N + base, RPT)],
                          bufA.at[pl.ds(base, RPT)], sem_i)
    d2 = pltpu.async_copy(h_hbm.at[pl.ds(c * N + base, RPT)],
                          bufB.at[pl.ds(base, RPT)], sem_i)
    d1.wait()
    d2.wait()
    _edge_phase(srcA_hbm, dstA_hbm, s, bufA, bufB,
                idx_s, idx_d, rows, sem_g, sem_s, sem_i)
    plsc.subcore_barrier()
    for q in range(PQ):
        pltpu.sync_copy(bufB.at[pl.ds(base + q * PK, PK)], stage)
        pltpu.sync_copy(stage, bufA.at[pl.ds(base + q * PK, PK)])
    _edge_phase(srcB_hbm, dstB_hbm, s, bufB, bufA,
                idx_s, idx_d, rows, sem_g, sem_s, sem_i)
    plsc.subcore_barrier()
    pltpu.sync_copy(bufA.at[pl.ds(base, RPT)],
                    out_hbm.at[pl.ds(c * N + base, RPT)])


_agg2 = functools.partial(
    pl.kernel,
    _agg2_impl,
    out_type=jax.ShapeDtypeStruct((2 * N, HALF), jnp.float32),
    mesh=_mesh,
    scratch_types=[
        pltpu.VMEM_SHARED((ACC_ROWS, HALF), jnp.float32),
        pltpu.VMEM_SHARED((ACC_ROWS, HALF), jnp.float32),
        pltpu.VMEM((3, NIDX, K), jnp.int32),
        pltpu.VMEM((3, NIDX, K), jnp.int32),
        pltpu.VMEM((4, K, HALF), jnp.float32),
        pltpu.VMEM((PK, HALF), jnp.float32),
        pltpu.SemaphoreType.DMA,
        pltpu.SemaphoreType.DMA,
        pltpu.SemaphoreType.DMA,
    ],
    compiler_params=_sc_params,
)()


_agg = functools.partial(
    pl.kernel,
    functools.partial(_agg_impl, False),
    out_type=jax.ShapeDtypeStruct((2 * N, HALF), jnp.float32),
    mesh=_mesh,
    scratch_types=[
        pltpu.VMEM_SHARED((ACC_ROWS, HALF), jnp.float32),
        pltpu.VMEM_SHARED((N, HALF), jnp.float32),
        pltpu.VMEM((3, NIDX, K), jnp.int32),
        pltpu.VMEM((3, NIDX, K), jnp.int32),
        pltpu.VMEM((4, K, HALF), jnp.float32),
        pltpu.VMEM((PK, HALF), jnp.float32),
        pltpu.SemaphoreType.DMA,
        pltpu.SemaphoreType.DMA,
        pltpu.SemaphoreType.DMA,
    ],
    compiler_params=_sc_params,
)()

_agg_pool = functools.partial(
    pl.kernel,
    functools.partial(_agg_impl, True),
    out_type=jax.ShapeDtypeStruct((NC, G, HALF), jnp.float32),
    mesh=_mesh,
    scratch_types=[
        pltpu.VMEM_SHARED((ACC_ROWS, HALF), jnp.float32),
        pltpu.VMEM_SHARED((N, HALF), jnp.float32),
        pltpu.VMEM((3, NIDX, K), jnp.int32),
        pltpu.VMEM((3, NIDX, K), jnp.int32),
        pltpu.VMEM((4, K, HALF), jnp.float32),
        pltpu.VMEM((PK, HALF), jnp.float32),
        pltpu.VMEM_SHARED((G, HALF), jnp.float32),
        pltpu.VMEM((PQ, PK), jnp.int32),
        pltpu.SemaphoreType.DMA,
        pltpu.SemaphoreType.DMA,
        pltpu.SemaphoreType.DMA,
    ],
    compiler_params=_sc_params,
)()


BLK = 2000


def _mm_body(h0_ref, h1_ref, wt_ref, b_ref, o_ref):
    h0 = h0_ref[...]
    h1 = h1_ref[...]
    wt = wt_ref[0]
    acc = lax.dot_general(h0, wt[:HALF], (((1,), (0,)), ((), ())),
                          preferred_element_type=jnp.float32)
    acc += lax.dot_general(h1, wt[HALF:], (((1,), (0,)), ((), ())),
                           preferred_element_type=jnp.float32)
    o_ref[...] = jnp.maximum(acc + b_ref[0], 0.0)


_mm = pl.pallas_call(
    _mm_body,
    grid=(2, N // BLK),
    in_specs=[
        pl.BlockSpec((BLK, HALF), lambda half, i: (i, 0)),
        pl.BlockSpec((BLK, HALF), lambda half, i: (N // BLK + i, 0)),
        pl.BlockSpec((1, D, HALF), lambda half, i: (half, 0, 0)),
        pl.BlockSpec((1, 1, HALF), lambda half, i: (half, 0, 0)),
    ],
    out_specs=pl.BlockSpec((BLK, HALF), lambda half, i: (half * (N // BLK) + i, 0)),
    out_shape=jax.ShapeDtypeStruct((2 * N, HALF), jnp.float32),
)


def _prep_edges(ei):
    src, dst = ei[0], ei[1]
    pad = NS * NCHUNK * K - E
    src = jnp.concatenate([src, jnp.zeros((pad,), jnp.int32)])
    dst = jnp.concatenate([dst, jnp.full((pad,), N, jnp.int32)])
    return (src.reshape(NS, NBLK, NIDX, K), dst.reshape(NS, NBLK, NIDX, K))


def kernel(x, edge_index, expander_edge_index, batch, W1, b1, W2, b2, W3, b3):
    h = jnp.concatenate([x[:, :HALF], x[:, HALF:]], axis=0)
    src_e, dst_e = _prep_edges(edge_index)
    src_x, dst_x = _prep_edges(expander_edge_index)
    batch_i = batch.reshape(NS, PQ, PK)
    zero = jnp.zeros((G, HALF), jnp.float32)
    h = _agg(h, src_e, dst_e)
    for li, (W, b) in enumerate(((W1, b1), (W2, b2), (W3, b3))):
        wt = W.T.reshape(1, D, D)
        wt = jnp.concatenate([wt[:, :, :HALF], wt[:, :, HALF:]], axis=0)
        h = _mm(h, h, wt, b.reshape(2, 1, HALF))
        if li < 2:
            h = _agg2(h, src_x, dst_x, src_e, dst_e)
        else:
            pooled = _agg_pool(h, src_x, dst_x, batch_i, zero)
    return jnp.concatenate([pooled[0], pooled[1]], axis=1).reshape(-1)


# speedup vs baseline: 2.3178x; 1.0173x over previous
import functools

import jax
import jax.numpy as jnp
from jax import lax
from jax.experimental import pallas as pl
from jax.experimental.pallas import tpu as pltpu
from jax.experimental.pallas import tpu_sc as plsc

N = 10000
D = 128
E = 320000
G = 64
HALF = 64

NC = 2
NS = 16
K = 128
NIDX = 8
NBLK = 20
NCHUNK = NBLK * NIDX
NGRP = NCHUNK // 2
EPT = NCHUNK * K
RPT = N // NS
ACC_ROWS = N + 16
PQ = 5
PK = RPT // PQ

_mesh = plsc.VectorSubcoreMesh(core_axis_name="c", subcore_axis_name="s")
_sc_params = pltpu.CompilerParams(use_tc_tiling_on_sc=False)


def _edge_phase(src_hbm, dst_hbm, s, gbuf, acc,
                idx_s, idx_d, rows, sem_g, sem_s, sem_i):
    for blk in range(2):
        pltpu.async_copy(src_hbm.at[s, blk], idx_s.at[blk], sem_i)
        pltpu.async_copy(dst_hbm.at[s, blk], idx_d.at[blk], sem_i)
    plsc.subcore_barrier()

    def body(i, carry):
        @pl.when(i % 2 == 0)
        def _():
            pltpu.make_async_copy(src_hbm.at[s, 0], idx_s.at[0],
                                  sem_i).wait()
            pltpu.make_async_copy(dst_hbm.at[s, 0], idx_d.at[0],
                                  sem_i).wait()

        @pl.when(jnp.logical_and(i % 2 == 1, i // 2 + 2 < NBLK))
        def _():
            b2 = i // 2 + 2
            slot2 = b2 % 3
            pltpu.async_copy(src_hbm.at[s, b2], idx_s.at[slot2], sem_i)
            pltpu.async_copy(dst_hbm.at[s, b2], idx_d.at[slot2], sem_i)

        descs = []
        for u in range(4):
            j = 4 * i + u
            slot = (j // NIDX) % 3
            cb = j % NIDX

            @pl.when(i > 0)
            def _():
                pltpu.make_async_copy(rows.at[u], acc.at[idx_d.at[0, 0]],
                                      sem_s).wait()

            descs.append(pltpu.async_copy(gbuf.at[idx_s.at[slot, cb]],
                                          rows.at[u], sem_g))
        for u in range(4):
            j = 4 * i + u
            slot = (j // NIDX) % 3
            cb = j % NIDX
            descs[u].wait()
            pltpu.async_copy(rows.at[u], acc.at[idx_d.at[slot, cb]],
                             sem_s, add=True)
        return carry

    lax.fori_loop(0, NCHUNK // 4, body, 0)
    for u in range(4):
        pltpu.make_async_copy(rows.at[u], acc.at[idx_d.at[0, 0]],
                              sem_s).wait()


def _agg_impl(do_pool, h_hbm, src_hbm, dst_hbm, *rest):
    if do_pool:
        (batch_hbm, zero_hbm, pool_out, acc, h_spm,
         idx_s, idx_d, rows, stage, pooled, bidx, sem_g, sem_s, sem_i) = rest
    else:
        (out_hbm, acc, h_spm,
         idx_s, idx_d, rows, stage, sem_g, sem_s, sem_i) = rest
    c = lax.axis_index("c")
    s = lax.axis_index("s")
    base = s * RPT

    d1 = pltpu.async_copy(h_hbm.at[pl.ds(c * N + base, RPT)],
                          acc.at[pl.ds(base, RPT)], sem_i)
    d2 = pltpu.async_copy(h_hbm.at[pl.ds(c * N + base, RPT)],
                          h_spm.at[pl.ds(base, RPT)], sem_i)
    d1.wait()
    d2.wait()
    _edge_phase(src_hbm, dst_hbm, s, h_spm, acc,
                idx_s, idx_d, rows, sem_g, sem_s, sem_i)
    plsc.subcore_barrier()

    if not do_pool:
        pltpu.sync_copy(acc.at[pl.ds(base, RPT)],
                        out_hbm.at[pl.ds(c * N + base, RPT)])
        return

    @pl.when(s == 0)
    def _():
        pltpu.sync_copy(zero_hbm, pooled)
    plsc.subcore_barrier()
    pltpu.sync_copy(batch_hbm.at[s], bidx)
    for q in range(PQ):
        pltpu.sync_copy(acc.at[pl.ds(base + q * PK, PK)], stage)
        pltpu.sync_copy(stage, pooled.at[bidx.at[q]], add=True)
    plsc.subcore_barrier()

    @pl.when(s == 0)
    def _():
        pltpu.sync_copy(pooled, stage.at[pl.ds(0, G)])
        pltpu.sync_copy(stage.at[pl.ds(0, G)], pool_out.at[c])


def _agg2_impl(h_hbm, srcA_hbm, dstA_hbm, srcB_hbm, dstB_hbm, out_hbm,
               bufA, bufB, idx_s, idx_d, rows, stage, sem_g, sem_s, sem_i):
    c = lax.axis_index("c")
    s = lax.axis_index("s")
    base = s * RPT

    d1 = pltpu.async_copy(h_hbm.at[pl.ds(c * N + base, RPT)],
                          bufA.at[pl.ds(base, RPT)], sem_i)
    d2 = pltpu.async_copy(h_hbm.at[pl.ds(c * N + base, RPT)],
                          bufB.at[pl.ds(base, RPT)], sem_i)
    d1.wait()
    d2.wait()
    _edge_phase(srcA_hbm, dstA_hbm, s, bufA, bufB,
                idx_s, idx_d, rows, sem_g, sem_s, sem_i)
    plsc.subcore_barrier()
    for q in range(PQ):
        pltpu.sync_copy(bufB.at[pl.ds(base + q * PK, PK)], stage)
        pltpu.sync_copy(stage, bufA.at[pl.ds(base + q * PK, PK)])
    _edge_phase(srcB_hbm, dstB_hbm, s, bufB, bufA,
                idx_s, idx_d, rows, sem_g, sem_s, sem_i)
    plsc.subcore_barrier()
    pltpu.sync_copy(bufA.at[pl.ds(base, RPT)],
                    out_hbm.at[pl.ds(c * N + base, RPT)])


_agg2 = functools.partial(
    pl.kernel,
    _agg2_impl,
    out_type=jax.ShapeDtypeStruct((2 * N, HALF), jnp.float32),
    mesh=_mesh,
    scratch_types=[
        pltpu.VMEM_SHARED((ACC_ROWS, HALF), jnp.float32),
        pltpu.VMEM_SHARED((ACC_ROWS, HALF), jnp.float32),
        pltpu.VMEM((3, NIDX, K), jnp.int32),
        pltpu.VMEM((3, NIDX, K), jnp.int32),
        pltpu.VMEM((4, K, HALF), jnp.float32),
        pltpu.VMEM((PK, HALF), jnp.float32),
        pltpu.SemaphoreType.DMA,
        pltpu.SemaphoreType.DMA,
        pltpu.SemaphoreType.DMA,
    ],
    compiler_params=_sc_params,
)()


_agg = functools.partial(
    pl.kernel,
    functools.partial(_agg_impl, False),
    out_type=jax.ShapeDtypeStruct((2 * N, HALF), jnp.float32),
    mesh=_mesh,
    scratch_types=[
        pltpu.VMEM_SHARED((ACC_ROWS, HALF), jnp.float32),
        pltpu.VMEM_SHARED((N, HALF), jnp.float32),
        pltpu.VMEM((3, NIDX, K), jnp.int32),
        pltpu.VMEM((3, NIDX, K), jnp.int32),
        pltpu.VMEM((4, K, HALF), jnp.float32),
        pltpu.VMEM((PK, HALF), jnp.float32),
        pltpu.SemaphoreType.DMA,
        pltpu.SemaphoreType.DMA,
        pltpu.SemaphoreType.DMA,
    ],
    compiler_params=_sc_params,
)()

_agg_pool = functools.partial(
    pl.kernel,
    functools.partial(_agg_impl, True),
    out_type=jax.ShapeDtypeStruct((NC, G, HALF), jnp.float32),
    mesh=_mesh,
    scratch_types=[
        pltpu.VMEM_SHARED((ACC_ROWS, HALF), jnp.float32),
        pltpu.VMEM_SHARED((N, HALF), jnp.float32),
        pltpu.VMEM((3, NIDX, K), jnp.int32),
        pltpu.VMEM((3, NIDX, K), jnp.int32),
        pltpu.VMEM((4, K, HALF), jnp.float32),
        pltpu.VMEM((PK, HALF), jnp.float32),
        pltpu.VMEM_SHARED((G, HALF), jnp.float32),
        pltpu.VMEM((PQ, PK), jnp.int32),
        pltpu.SemaphoreType.DMA,
        pltpu.SemaphoreType.DMA,
        pltpu.SemaphoreType.DMA,
    ],
    compiler_params=_sc_params,
)()


BLK = 2000


def _mm_body(h0_ref, h1_ref, wt_ref, b_ref, o_ref):
    h0 = h0_ref[...]
    h1 = h1_ref[...]
    wt = wt_ref[0]
    acc = lax.dot_general(h0, wt[:HALF], (((1,), (0,)), ((), ())),
                          preferred_element_type=jnp.float32)
    acc += lax.dot_general(h1, wt[HALF:], (((1,), (0,)), ((), ())),
                           preferred_element_type=jnp.float32)
    o_ref[...] = jnp.maximum(acc + b_ref[0], 0.0)


_mm = pl.pallas_call(
    _mm_body,
    grid=(2,),
    in_specs=[
        pl.BlockSpec((N, HALF), lambda half: (0, 0)),
        pl.BlockSpec((N, HALF), lambda half: (1, 0)),
        pl.BlockSpec((1, D, HALF), lambda half: (half, 0, 0)),
        pl.BlockSpec((1, 1, HALF), lambda half: (half, 0, 0)),
    ],
    out_specs=pl.BlockSpec((N, HALF), lambda half: (half, 0)),
    out_shape=jax.ShapeDtypeStruct((2 * N, HALF), jnp.float32),
)


def _prep_edges(ei):
    src, dst = ei[0], ei[1]
    pad = NS * NCHUNK * K - E
    src = jnp.concatenate([src, jnp.zeros((pad,), jnp.int32)])
    dst = jnp.concatenate([dst, jnp.full((pad,), N, jnp.int32)])
    return (src.reshape(NS, NBLK, NIDX, K), dst.reshape(NS, NBLK, NIDX, K))


def kernel(x, edge_index, expander_edge_index, batch, W1, b1, W2, b2, W3, b3):
    h = jnp.concatenate([x[:, :HALF], x[:, HALF:]], axis=0)
    src_e, dst_e = _prep_edges(edge_index)
    src_x, dst_x = _prep_edges(expander_edge_index)
    batch_i = batch.reshape(NS, PQ, PK)
    zero = jnp.zeros((G, HALF), jnp.float32)
    h = _agg(h, src_e, dst_e)
    for li, (W, b) in enumerate(((W1, b1), (W2, b2), (W3, b3))):
        wt = W.T.reshape(1, D, D)
        wt = jnp.concatenate([wt[:, :, :HALF], wt[:, :, HALF:]], axis=0)
        h = _mm(h, h, wt, b.reshape(2, 1, HALF))
        if li < 2:
            h = _agg2(h, src_x, dst_x, src_e, dst_e)
        else:
            pooled = _agg_pool(h, src_x, dst_x, batch_i, zero)
    return jnp.concatenate([pooled[0], pooled[1]], axis=1).reshape(-1)
